# baseline (speedup 1.0000x reference)
import functools

import jax
import jax.numpy as jnp
from jax import lax
from jax.experimental import pallas as pl
from jax.experimental.pallas import tpu as pltpu
from jax.experimental.pallas import tpu_sc as plsc

_NC = 2
_NS = 16
_NW = _NC * _NS
_C = 256


@jax.jit
def _gather(pidx, hoff, table2):
    J, B0 = 26, 16384
    D = 64
    b_per_w = (J * B0) // _NW
    bw = B0 // _NW
    n_chunks = b_per_w // _C

    mesh = plsc.VectorSubcoreMesh(core_axis_name="c", subcore_axis_name="s")

    @functools.partial(
        pl.kernel,
        mesh=mesh,
        compiler_params=pltpu.CompilerParams(
            use_tc_tiling_on_sc=True, needs_layout_passes=False
        ),
        out_type=jax.ShapeDtypeStruct((J, D, B0), jnp.float32),
        scratch_types=[
            pltpu.VMEM((b_per_w,), jnp.int32),
            pltpu.VMEM((b_per_w,), jnp.int32),
            pltpu.VMEM((2, _C, 2 * D), jnp.float32),
            pltpu.VMEM((D, _C), jnp.float32),
            pltpu.SemaphoreType.DMA,
            pltpu.SemaphoreType.DMA,
        ],
    )
    def k(pidx_hbm, hoff_hbm, table_hbm, out_hbm,
          pidx_v, hoff_v, pair_v, blk_v, gsem0, gsem1):
        wid = lax.axis_index("s") * _NC + lax.axis_index("c")
        base = wid * b_per_w
        pltpu.sync_copy(pidx_hbm.at[pl.ds(base, b_per_w)], pidx_v)
        pltpu.sync_copy(hoff_hbm.at[pl.ds(base, b_per_w)], hoff_v)
        gsems = (gsem0, gsem1)

        def fire(chunk, slot):
            pltpu.async_copy(
                table_hbm.at[pidx_v.at[pl.ds(chunk * _C, _C)]],
                pair_v.at[slot],
                gsems[slot],
            )

        def wait_gather(chunk, slot):
            pltpu.make_async_copy(
                table_hbm.at[pidx_v.at[pl.ds(chunk * _C, _C)]],
                pair_v.at[slot],
                gsems[slot],
            ).wait()

        def transpose_select(chunk, slot):
            @pl.loop(0, _C // 16)
            def _(cb):
                row16 = cb * 16 + lax.iota(jnp.int32, 16)
                col16 = hoff_v[pl.ds(chunk * _C + cb * 16, 16)]
                for d in range(D):
                    v = plsc.load_gather(pair_v.at[slot], [row16, col16 + d])
                    blk_v[d, pl.ds(cb * 16, 16)] = v

        def writeback(chunk):
            j = chunk // 2
            half = lax.rem(chunk, 2)
            col0 = wid * bw + half * _C
            pltpu.sync_copy(blk_v, out_hbm.at[j, :, pl.ds(col0, _C)])

        fire(0, 0)
        fire(1, 1)

        @pl.loop(0, n_chunks, step=2)
        def _(i):
            for b in range(2):
                chunk = i + b
                wait_gather(chunk, b)
                transpose_select(chunk, b)

                @pl.when(chunk + 2 < n_chunks)
                def _():
                    fire(chunk + 2, b)

                writeback(chunk)

    return k(pidx, hoff, table2)


def kernel(inputs, table):
    B0, B1 = inputs.shape
    V, D = table.shape
    idxw = (
        inputs.T.reshape(B1, _NW, B0 // _NW)
        .transpose(1, 0, 2)
        .reshape(-1)
    )
    pidx = (idxw >> 1).astype(jnp.int32)
    hoff = ((idxw & 1) << 6).astype(jnp.int32)
    table2 = table.reshape(V // 2, 2 * D)
    out3 = _gather(pidx, hoff, table2)
    return jnp.transpose(out3, (2, 0, 1))


# speedup vs baseline: 1.3447x; 1.3447x over previous
import functools

import jax
import jax.numpy as jnp
from jax import lax
from jax.experimental import pallas as pl
from jax.experimental.pallas import tpu as pltpu
from jax.experimental.pallas import tpu_sc as plsc

_NC = 2
_NS = 16
_NW = _NC * _NS


@functools.partial(jax.jit, static_argnames=("C",))
def _gather(idx, table, C):
    B, = idx.shape
    V, D = table.shape
    b_per_w = B // _NW
    n_chunks = b_per_w // C
    assert b_per_w % C == 0 and n_chunks % 2 == 0

    mesh = plsc.VectorSubcoreMesh(core_axis_name="c", subcore_axis_name="s")

    @functools.partial(
        pl.kernel,
        mesh=mesh,
        compiler_params=pltpu.CompilerParams(use_tc_tiling_on_sc=False),
        out_type=jax.ShapeDtypeStruct((B, D), jnp.float32),
        scratch_types=[
            pltpu.VMEM((b_per_w,), jnp.int32),
            pltpu.VMEM((2, C, D), jnp.float32),
            pltpu.SemaphoreType.DMA,
            pltpu.SemaphoreType.DMA,
        ],
    )
    def k(idx_hbm, table_hbm, out_hbm, idx_v, rows_v, gsem0, gsem1):
        wid = lax.axis_index("s") * _NC + lax.axis_index("c")
        base = wid * b_per_w
        pltpu.sync_copy(idx_hbm.at[pl.ds(base, b_per_w)], idx_v)
        gsems = (gsem0, gsem1)

        def fire(chunk, slot):
            pltpu.async_copy(
                table_hbm.at[idx_v.at[pl.ds(chunk * C, C)]],
                rows_v.at[slot],
                gsems[slot],
            )

        def wait_writeback(chunk, slot):
            pltpu.make_async_copy(
                table_hbm.at[idx_v.at[pl.ds(chunk * C, C)]],
                rows_v.at[slot],
                gsems[slot],
            ).wait()
            pltpu.sync_copy(
                rows_v.at[slot], out_hbm.at[pl.ds(base + chunk * C, C)]
            )

        fire(0, 0)
        fire(1, 1)

        @pl.loop(0, n_chunks, step=2)
        def _(i):
            for b in range(2):
                chunk = i + b
                wait_writeback(chunk, b)

                @pl.when(chunk + 2 < n_chunks)
                def _():
                    fire(chunk + 2, b)

    return k(idx, table)


def kernel(inputs, table):
    B0, B1 = inputs.shape
    _, D = table.shape
    idx = inputs.reshape(B0 * B1).astype(jnp.int32)
    out = _gather(idx, table, C=512)
    return out.reshape(B0, B1, D)
